# Initial kernel scaffold; baseline (speedup 1.0000x reference)
import jax
import jax.numpy as jnp
from jax.experimental import pallas as pl


def kernel(z, pos, batch, emb_table, mlp_w1, mlp_b1, mlp_w2, mlp_b2, lin1_w, lin2_w, lin2_b, lin_w, lin_b, pool_w, pool_b):
    raise NotImplementedError("write your pallas kernel here")


# speedup vs baseline: 10.2663x; 10.2663x over previous
import jax
import jax.numpy as jnp
from jax.experimental import pallas as pl

G = 128
P = 64
HIDDEN = 128
NF = 128
NG = 50
NI = 6
CUTOFF = 10.0
EMB = 128


def _ssp(x):
    return jnp.maximum(x, 0.0) + jnp.log1p(jnp.exp(-jnp.abs(x))) - 0.6931471805599453


def _mol_kernel(z_ref, pos_ref, emb_ref, w1_ref, b1_ref, w2_ref, b2_ref,
                l1_ref, l2w_ref, l2b_ref, lw_ref, lb_ref, pw_ref, pb_ref,
                out_ref):
    p = pos_ref[0]
    zc = z_ref[0]

    tt = jax.lax.broadcasted_iota(jnp.int32, (P, 100), 1)
    oh = (zc == tt).astype(jnp.float32)
    h = jnp.dot(oh, emb_ref[...], preferred_element_type=jnp.float32)

    diff = p[:, None, :] - p[None, :, :]
    r2 = jnp.sum(diff * diff, axis=-1, keepdims=True)
    ewf = jnp.sqrt(r2 + 1e-12).reshape(P * P, 1)

    step = CUTOFF / (NG - 1)
    offs = jax.lax.broadcasted_iota(jnp.int32, (1, NG), 1).astype(jnp.float32) * step
    gcoeff = -0.5 / (step * step)
    ea = jnp.exp(gcoeff * (ewf - offs) ** 2)

    cf = 0.5 * (jnp.cos(ewf * (jnp.pi / CUTOFF)) + 1.0)
    cf = cf * (ewf < CUTOFF).astype(jnp.float32)
    ii = jax.lax.broadcasted_iota(jnp.int32, (P, P, 1), 0)
    jj = jax.lax.broadcasted_iota(jnp.int32, (P, P, 1), 1)
    scale = cf * (ii != jj).astype(jnp.float32).reshape(P * P, 1)

    for t in range(NI):
        h1 = _ssp(jnp.dot(ea, w1_ref[t], preferred_element_type=jnp.float32)
                  + b1_ref[t])
        w = jnp.dot(h1, w2_ref[t], preferred_element_type=jnp.float32) + b2_ref[t]
        w = w * scale
        x = jnp.dot(h, l1_ref[t], preferred_element_type=jnp.float32)
        w3 = w.reshape(P, P, NF)
        agg = jnp.sum(w3 * x[:, None, :], axis=0)
        x2 = _ssp(jnp.dot(agg, l2w_ref[t], preferred_element_type=jnp.float32)
                  + l2b_ref[t])
        x2 = jnp.dot(x2, lw_ref[t], preferred_element_type=jnp.float32) + lb_ref[t]
        h = h + x2

    hm = jnp.sum(h, axis=0, keepdims=True) * (1.0 / P)
    out_ref[0] = jnp.dot(hm, pw_ref[...], preferred_element_type=jnp.float32) \
        + pb_ref[...]


def kernel(z, pos, batch, emb_table, mlp_w1, mlp_b1, mlp_w2, mlp_b2,
           lin1_w, lin2_w, lin2_b, lin_w, lin_b, pool_w, pool_b):
    del batch
    z3 = z.reshape(G, P, 1).astype(jnp.int32)
    pos3 = pos.reshape(G, P, 3)
    b1 = mlp_b1.reshape(NI, 1, NF)
    b2 = mlp_b2.reshape(NI, 1, NF)
    l2b = lin2_b.reshape(NI, 1, HIDDEN)
    lb = lin_b.reshape(NI, 1, HIDDEN)
    pb = pool_b.reshape(1, EMB)

    def whole(a):
        return pl.BlockSpec(a.shape, lambda g: (0,) * a.ndim)

    out = pl.pallas_call(
        _mol_kernel,
        grid=(G,),
        in_specs=[
            pl.BlockSpec((1, P, 1), lambda g: (g, 0, 0)),
            pl.BlockSpec((1, P, 3), lambda g: (g, 0, 0)),
            whole(emb_table),
            whole(mlp_w1), whole(b1), whole(mlp_w2), whole(b2),
            whole(lin1_w), whole(lin2_w), whole(l2b),
            whole(lin_w), whole(lb), whole(pool_w), whole(pb),
        ],
        out_specs=pl.BlockSpec((1, 1, EMB), lambda g: (g, 0, 0)),
        out_shape=jax.ShapeDtypeStruct((G, 1, EMB), jnp.float32),
    )(z3, pos3, emb_table, mlp_w1, b1, mlp_w2, b2,
      lin1_w, lin2_w, l2b, lin_w, lb, pool_w, pb)
    return out.reshape(G, EMB)
